# baseline (speedup 1.0000x reference)
import jax
import jax.numpy as jnp
from jax.experimental import pallas as pl
from jax.experimental.pallas import tpu as pltpu

_C, _H, _W = 64, 512, 512
_OH, _OW = 256, 256
_BR = 1024
_SR = 512


def _pm_even(v):
    r = pltpu.roll(v, 127, 1)
    return jnp.maximum(v - r, 0.0) + jnp.maximum(r, 0.0)


def _pm_odd(v):
    r = pltpu.roll(v, 1, 1)
    return jnp.maximum(r - v, 0.0) + jnp.maximum(v, 0.0)


def _pool_block(x_ref, o_ref):
    lane = jax.lax.broadcasted_iota(jnp.int32, (_SR, 128), 1)
    parity = (lane & 1) == 0
    idx = jnp.where(lane < 64, 2 * lane, 2 * lane - 127)
    for s in range(_BR // _SR):
        r0 = s * _SR
        for t in range(2):
            k0, k1 = 2 * t, 2 * t + 1
            m0 = jnp.maximum(
                _pm_even(x_ref[r0 : r0 + _SR, 128 * k0 : 128 * k0 + 128]),
                _pm_even(x_ref[r0 : r0 + _SR, 512 + 128 * k0 : 640 + 128 * k0]),
            )
            m1 = jnp.maximum(
                _pm_odd(x_ref[r0 : r0 + _SR, 128 * k1 : 128 * k1 + 128]),
                _pm_odd(x_ref[r0 : r0 + _SR, 512 + 128 * k1 : 640 + 128 * k1]),
            )
            z = jnp.where(parity, m0, m1)
            o_ref[r0 : r0 + _SR, 128 * t : 128 * t + 128] = (
                jnp.take_along_axis(z, idx, axis=1)
            )


def kernel(x):
    rows = _C * _H // 2
    x2 = x.reshape(rows, 2 * _W)
    out = pl.pallas_call(
        _pool_block,
        grid=(rows // _BR,),
        in_specs=[pl.BlockSpec((_BR, 2 * _W), lambda i: (i, 0))],
        out_specs=pl.BlockSpec((_BR, _OW), lambda i: (i, 0)),
        out_shape=jax.ShapeDtypeStruct((rows, _OW), x.dtype),
        compiler_params=pltpu.CompilerParams(
            dimension_semantics=("parallel",),
        ),
    )(x2)
    return out.reshape(_C, _OH, _OW)


# speedup vs baseline: 1.0260x; 1.0260x over previous
import jax
import jax.numpy as jnp
from jax.experimental import pallas as pl
from jax.experimental.pallas import tpu as pltpu

_C, _H, _W = 64, 512, 512
_OH, _OW = 256, 256
_BR = 2048
_SR = 512


def _pm_even(v):
    r = pltpu.roll(v, 127, 1)
    return jnp.maximum(v - r, 0.0) + jnp.maximum(r, 0.0)


def _pm_odd(v):
    r = pltpu.roll(v, 1, 1)
    return jnp.maximum(r - v, 0.0) + jnp.maximum(v, 0.0)


def _pool_block(x_ref, o_ref):
    lane = jax.lax.broadcasted_iota(jnp.int32, (_SR, 128), 1)
    parity = (lane & 1) == 0
    idx = jnp.where(lane < 64, 2 * lane, 2 * lane - 127)
    for s in range(_BR // _SR):
        r0 = s * _SR
        for t in range(2):
            k0, k1 = 2 * t, 2 * t + 1
            m0 = jnp.maximum(
                _pm_even(x_ref[r0 : r0 + _SR, 128 * k0 : 128 * k0 + 128]),
                _pm_even(x_ref[r0 : r0 + _SR, 512 + 128 * k0 : 640 + 128 * k0]),
            )
            m1 = jnp.maximum(
                _pm_odd(x_ref[r0 : r0 + _SR, 128 * k1 : 128 * k1 + 128]),
                _pm_odd(x_ref[r0 : r0 + _SR, 512 + 128 * k1 : 640 + 128 * k1]),
            )
            z = jnp.where(parity, m0, m1)
            o_ref[r0 : r0 + _SR, 128 * t : 128 * t + 128] = (
                jnp.take_along_axis(z, idx, axis=1)
            )


def kernel(x):
    rows = _C * _H // 2
    x2 = x.reshape(rows, 2 * _W)
    out = pl.pallas_call(
        _pool_block,
        grid=(rows // _BR,),
        in_specs=[pl.BlockSpec((_BR, 2 * _W), lambda i: (i, 0))],
        out_specs=pl.BlockSpec((_BR, _OW), lambda i: (i, 0)),
        out_shape=jax.ShapeDtypeStruct((rows, _OW), x.dtype),
        compiler_params=pltpu.CompilerParams(
            dimension_semantics=("parallel",),
        ),
    )(x2)
    return out.reshape(_C, _OH, _OW)


# speedup vs baseline: 1.0316x; 1.0054x over previous
import jax
import jax.numpy as jnp
from jax.experimental import pallas as pl
from jax.experimental.pallas import tpu as pltpu

_C, _H, _W = 64, 512, 512
_OH, _OW = 256, 256
_BR = 4096
_SR = 512


def _pm_even(v):
    r = pltpu.roll(v, 127, 1)
    return jnp.maximum(v - r, 0.0) + jnp.maximum(r, 0.0)


def _pm_odd(v):
    r = pltpu.roll(v, 1, 1)
    return jnp.maximum(r - v, 0.0) + jnp.maximum(v, 0.0)


def _pool_block(x_ref, o_ref):
    lane = jax.lax.broadcasted_iota(jnp.int32, (_SR, 128), 1)
    parity = (lane & 1) == 0
    idx = jnp.where(lane < 64, 2 * lane, 2 * lane - 127)
    for s in range(_BR // _SR):
        r0 = s * _SR
        for t in range(2):
            k0, k1 = 2 * t, 2 * t + 1
            m0 = jnp.maximum(
                _pm_even(x_ref[r0 : r0 + _SR, 128 * k0 : 128 * k0 + 128]),
                _pm_even(x_ref[r0 : r0 + _SR, 512 + 128 * k0 : 640 + 128 * k0]),
            )
            m1 = jnp.maximum(
                _pm_odd(x_ref[r0 : r0 + _SR, 128 * k1 : 128 * k1 + 128]),
                _pm_odd(x_ref[r0 : r0 + _SR, 512 + 128 * k1 : 640 + 128 * k1]),
            )
            z = jnp.where(parity, m0, m1)
            o_ref[r0 : r0 + _SR, 128 * t : 128 * t + 128] = (
                jnp.take_along_axis(z, idx, axis=1)
            )


def kernel(x):
    rows = _C * _H // 2
    x2 = x.reshape(rows, 2 * _W)
    out = pl.pallas_call(
        _pool_block,
        grid=(rows // _BR,),
        in_specs=[pl.BlockSpec((_BR, 2 * _W), lambda i: (i, 0))],
        out_specs=pl.BlockSpec((_BR, _OW), lambda i: (i, 0)),
        out_shape=jax.ShapeDtypeStruct((rows, _OW), x.dtype),
        compiler_params=pltpu.CompilerParams(
            dimension_semantics=("parallel",),
        ),
    )(x2)
    return out.reshape(_C, _OH, _OW)
